# baseline (speedup 1.0000x reference)
import functools

import jax
import jax.numpy as jnp
from jax import lax
from jax.experimental import pallas as pl
from jax.experimental.pallas import tpu as pltpu
from jax.experimental.pallas import tpu_sc as plsc

BATCH = 16384
HIDDEN = 1024
NUM_CORES = 2
NUM_SUBCORES = 16
NUM_WORKERS = NUM_CORES * NUM_SUBCORES
ROWS_PER_WORKER = BATCH // NUM_WORKERS
CHUNK = 16
NUM_CHUNKS = ROWS_PER_WORKER // CHUNK

_mesh = plsc.VectorSubcoreMesh(core_axis_name="c", subcore_axis_name="s")


NBUF = 6
PRIME = NBUF - 1


@functools.partial(
    pl.kernel,
    mesh=_mesh,
    out_type=jax.ShapeDtypeStruct((BATCH, 1, HIDDEN), jnp.float32),
    scratch_types=[
        pltpu.VMEM((NUM_CHUNKS, CHUNK), jnp.int32),
        pltpu.VMEM((NBUF, CHUNK, 1, HIDDEN), jnp.float32),
        pltpu.SemaphoreType.DMA,
        pltpu.SemaphoreType.DMA,
    ],
)
def _embed_sc(x_hbm, table_hbm, out_hbm, idx_v, rows_v, gsem, wsem):
    wid = lax.axis_index("s") * NUM_CORES + lax.axis_index("c")
    pltpu.sync_copy(x_hbm.at[wid], idx_v)
    base = wid * ROWS_PER_WORKER

    def gather(c):
        return pltpu.async_copy(
            table_hbm.at[idx_v.at[c]], rows_v.at[c % NBUF], gsem)

    def writeback(c):
        return pltpu.async_copy(
            rows_v.at[c % NBUF],
            out_hbm.at[pl.ds(base + c * CHUNK, CHUNK)], wsem)

    copies = {}
    for c in range(PRIME):
        copies["g", c] = gather(c)
    for c in range(NUM_CHUNKS):
        nxt = c + PRIME
        if nxt < NUM_CHUNKS:
            if nxt >= NBUF:
                copies["w", nxt - NBUF].wait()
            copies["g", nxt] = gather(nxt)
        copies["g", c].wait()
        copies["w", c] = writeback(c)
    for c in range(NUM_CHUNKS - NBUF, NUM_CHUNKS):
        copies["w", c].wait()


_REPL = 128


def kernel(x, table):
    table_rep = jnp.tile(table, (_REPL, 1)).reshape(2 * _REPL, 1, HIDDEN)
    x_spread = x + 2 * (jnp.arange(BATCH, dtype=jnp.int32) % _REPL)
    xr = x_spread.reshape(NUM_WORKERS, NUM_CHUNKS, CHUNK)
    return _embed_sc(xr, table_rep)


# speedup vs baseline: 1.7177x; 1.7177x over previous
import functools

import jax
import jax.numpy as jnp
from jax import lax
from jax.experimental import pallas as pl
from jax.experimental.pallas import tpu as pltpu
from jax.experimental.pallas import tpu_sc as plsc

BATCH = 16384
HIDDEN = 1024
LANES = 16
NUM_CORES = 2
NUM_SUBCORES = 16
NUM_WORKERS = NUM_CORES * NUM_SUBCORES
ROWS_PER_WORKER = BATCH // NUM_WORKERS
GROUPS = ROWS_PER_WORKER // LANES

_mesh = plsc.VectorSubcoreMesh(core_axis_name="c", subcore_axis_name="s")


@functools.partial(
    pl.kernel,
    mesh=_mesh,
    out_type=jax.ShapeDtypeStruct((BATCH, 1, HIDDEN), jnp.float32),
    scratch_types=[
        pltpu.VMEM((GROUPS, LANES), jnp.int32),
        pltpu.VMEM((2, 1, HIDDEN), jnp.float32),
        pltpu.SemaphoreType.DMA,
    ],
)
def _embed_sc(x_hbm, table_hbm, out_hbm, idx_v, table_v, wsem):
    wid = lax.axis_index("s") * NUM_CORES + lax.axis_index("c")
    pltpu.sync_copy(x_hbm.at[wid], idx_v)
    pltpu.sync_copy(table_hbm, table_v)
    base = wid * ROWS_PER_WORKER

    handles = {}
    for g in range(GROUPS):
        xv = idx_v[g]
        for j in range(LANES):
            h = pltpu.make_async_copy(
                table_v.at[xv[j]], out_hbm.at[base + g * LANES + j], wsem)
            h.start()
            handles[g, j] = h
        if g >= 2:
            for j in range(LANES):
                handles[g - 2, j].wait()
    for g in range(GROUPS - 2, GROUPS):
        for j in range(LANES):
            handles[g, j].wait()


def kernel(x, table):
    xr = x.reshape(NUM_WORKERS, GROUPS, LANES)
    return _embed_sc(xr, table.reshape(2, 1, HIDDEN))
